# Initial kernel scaffold; baseline (speedup 1.0000x reference)
import jax
import jax.numpy as jnp
from jax.experimental import pallas as pl


def kernel(q_sub, q_rel, q_tau, hidden, edges, n_node, old_nodes_new_idx, rela_embed, Ws_attn, Wr_attn, Wqr_attn_w, Wqr_attn_b, Wtau_attn, w_alpha_w, w_alpha_b, W_h, W_h_s, weight_t1, bias_t1, weight_t2, bias_t2):
    raise NotImplementedError("write your pallas kernel here")


# speedup vs baseline: 2.8148x; 2.8148x over previous
import functools

import jax
import jax.numpy as jnp
from jax import lax
from jax.experimental import pallas as pl
from jax.experimental.pallas import tpu as pltpu
from jax.experimental.pallas import tpu_sc as plsc

NC, NS, L = 2, 16, 16
B = 128
F32 = jnp.float32


def _tables_tc(hidden, rela, delta, wsa, wra, wqrw, wqrb, wtau, wt1, bt1, wt2, bt2):
    n = hidden.shape[0]
    nr = rela.shape[0]

    def body(h_r, r_r, d_r, wsa_r, wra_r, wqrw_r, wqrb_r, wtau_r,
             wt1_r, bt1_r, wt2_r, bt2_r, as_o, br_o, cf_o, dt_o, ht_o):
        d = d_r[...]
        ht = wt1_r[...] * d + bt1_r[...] + jnp.sin(wt2_r[...] * d + bt2_r[...])
        ht_o[...] = ht
        as_o[...] = jnp.dot(h_r[...], wsa_r[...], preferred_element_type=F32)
        br_o[...] = jnp.dot(r_r[...], wra_r[...], preferred_element_type=F32)
        cf_o[...] = jnp.dot(r_r[...], wqrw_r[...], preferred_element_type=F32) + wqrb_r[...]
        dt_o[...] = jnp.dot(ht, wtau_r[...], preferred_element_type=F32)

    return pl.pallas_call(
        body,
        out_shape=(
            jax.ShapeDtypeStruct((n, 64), F32),
            jax.ShapeDtypeStruct((nr, 64), F32),
            jax.ShapeDtypeStruct((nr, 64), F32),
            jax.ShapeDtypeStruct((n, 64), F32),
            jax.ShapeDtypeStruct((n, 128), F32),
        ),
        compiler_params=pltpu.CompilerParams(vmem_limit_bytes=100 * 1024 * 1024),
    )(hidden, rela, delta, wsa, wra, wqrw, wqrb, wtau, wt1, bt1, wt2, bt2)


def _alpha_sc(as_t, br_t, cf_t, dt_t, sub, rel, tau, ridx, qrel, wa, wab):
    e_total = sub.shape[0]
    nblk = e_total // B
    nw = NC * NS
    mesh = plsc.VectorSubcoreMesh(core_axis_name="c", subcore_axis_name="s",
                                  num_cores=NC, num_subcores=NS)

    @functools.partial(
        pl.kernel,
        out_type=jax.ShapeDtypeStruct((e_total,), F32),
        mesh=mesh,
        scratch_types=[
            pltpu.VMEM((B,), jnp.int32),
            pltpu.VMEM((B,), jnp.int32),
            pltpu.VMEM((B,), jnp.int32),
            pltpu.VMEM((B,), jnp.int32),
            pltpu.VMEM((B,), jnp.int32),
            pltpu.VMEM((B, 64), F32),
            pltpu.VMEM((B, 64), F32),
            pltpu.VMEM((B, 64), F32),
            pltpu.VMEM((B, 64), F32),
            pltpu.VMEM((B,), F32),
            pltpu.VMEM((64,), F32),
            pltpu.VMEM((16,), F32),
            pltpu.SemaphoreType.DMA,
        ],
        compiler_params=pltpu.CompilerParams(needs_layout_passes=False, use_tc_tiling_on_sc=False),
    )
    def alpha_kernel(as_h, br_h, cf_h, dt_h, sub_h, rel_h, tau_h, ridx_h,
                     qrel_h, wa_h, wab_h, alpha_o,
                     subv, relv, tauv, ridxv, qv, s_rows, r_rows, c_rows,
                     d_rows, abuf, wav, wabv, sem):
        cid = lax.axis_index("c")
        sid = lax.axis_index("s")
        w = sid * NC + cid
        pltpu.sync_copy(wa_h, wav)
        pltpu.sync_copy(wab_h, wabv)
        wa_regs = [wav[pl.ds(16 * j, 16)] for j in range(4)]
        wab0 = wabv[...][0]
        lane = lax.iota(jnp.int32, 16)

        nblk_w = (nblk - w + nw - 1) // nw

        def do_block(i, carry):
            blk = w + i * nw
            base = blk * B
            pltpu.sync_copy(sub_h.at[pl.ds(base, B)], subv)
            pltpu.sync_copy(rel_h.at[pl.ds(base, B)], relv)
            pltpu.sync_copy(tau_h.at[pl.ds(base, B)], tauv)
            pltpu.sync_copy(ridx_h.at[pl.ds(base, B)], ridxv)
            pltpu.async_copy(qrel_h.at[ridxv], qv, sem).wait()
            cp1 = pltpu.async_copy(as_h.at[subv], s_rows, sem)
            cp2 = pltpu.async_copy(br_h.at[relv], r_rows, sem)
            cp3 = pltpu.async_copy(dt_h.at[tauv], d_rows, sem)
            cp4 = pltpu.async_copy(cf_h.at[qv], c_rows, sem)
            cp1.wait(); cp2.wait(); cp3.wait(); cp4.wait()

            def group(g, gcarry):
                zv = jnp.zeros((16,), F32)
                for k in range(16):
                    e = g * 16 + k
                    z = wab0
                    for j in range(4):
                        dj = pl.ds(16 * j, 16)
                        y = (s_rows[e, dj] + r_rows[e, dj]
                             + c_rows[e, dj] + d_rows[e, dj])
                        y = jnp.maximum(y, 0.0)
                        z = z + jnp.sum(y * wa_regs[j])
                    zv = jnp.where(lane == k, z, zv)
                abuf[pl.ds(g * 16, 16)] = 1.0 / (1.0 + jnp.exp(-zv))
                return gcarry

            lax.fori_loop(0, B // 16, group, 0)
            pltpu.sync_copy(abuf, alpha_o.at[pl.ds(base, B)])
            return carry

        lax.fori_loop(0, nblk_w, do_block, 0)

    return alpha_kernel(as_t, br_t, cf_t, dt_t, sub, rel, tau, ridx, qrel, wa, wab)


def _aggregate_sc(hid4, rela4, ht4, alpha, sub4, rel4, tau4, obj, zrows, n):
    e_total = obj.shape[0]
    nblk = e_total // B
    rows_per_sub = n // NS
    mesh = plsc.VectorSubcoreMesh(core_axis_name="c", subcore_axis_name="s",
                                  num_cores=NC, num_subcores=NS)

    @functools.partial(
        pl.kernel,
        out_type=(
            jax.ShapeDtypeStruct((4, n, 32), F32),
            jax.ShapeDtypeStruct((4, n, 32), F32),
        ),
        mesh=mesh,
        scratch_types=[
            pltpu.VMEM((B,), jnp.int32),
            pltpu.VMEM((B,), jnp.int32),
            pltpu.VMEM((B,), jnp.int32),
            pltpu.VMEM((B,), jnp.int32),
            pltpu.VMEM((B,), F32),
            pltpu.VMEM((B, 32), F32),
            pltpu.VMEM((B, 32), F32),
            pltpu.VMEM((B, 32), F32),
            pltpu.VMEM((B, 32), F32),
            pltpu.VMEM((B, 32), F32),
            pltpu.VMEM((rows_per_sub, 32), F32),
            pltpu.VMEM_SHARED((n, 32), F32),
            pltpu.VMEM_SHARED((n, 32), F32),
            pltpu.SemaphoreType.DMA,
        ],
        compiler_params=pltpu.CompilerParams(needs_layout_passes=False, use_tc_tiling_on_sc=False),
    )
    def agg_kernel(hid_h, rela_h, ht_h, alpha_h, sub_h, rel_h, tau_h, obj_h,
                   z_h, accw_o, acct_o,
                   subv, relv, tauv, objv, av, s_rows, r_rows, t_rows,
                   mw, mt, zv, accw, acct, sem):
        cid = lax.axis_index("c")
        sid = lax.axis_index("s")
        pltpu.sync_copy(z_h, zv)
        sl = pl.ds(sid * rows_per_sub, rows_per_sub)

        for p in range(2):
            q = cid * 2 + p
            pltpu.sync_copy(zv, accw.at[sl])
            pltpu.sync_copy(zv, acct.at[sl])
            plsc.subcore_barrier()

            nblk_w = (nblk - sid + NS - 1) // NS

            def do_block(i, carry):
                blk = sid + i * NS
                base = blk * B
                ebase = q * e_total + base
                pltpu.sync_copy(sub_h.at[pl.ds(ebase, B)], subv)
                pltpu.sync_copy(rel_h.at[pl.ds(ebase, B)], relv)
                pltpu.sync_copy(tau_h.at[pl.ds(ebase, B)], tauv)
                pltpu.sync_copy(obj_h.at[pl.ds(base, B)], objv)
                pltpu.sync_copy(alpha_h.at[pl.ds(base, B)], av)
                cp1 = pltpu.async_copy(hid_h.at[subv], s_rows, sem)
                cp2 = pltpu.async_copy(rela_h.at[relv], r_rows, sem)
                cp3 = pltpu.async_copy(ht_h.at[tauv], t_rows, sem)
                cp1.wait(); cp2.wait(); cp3.wait()

                def group(g, gcarry):
                    a16 = av[pl.ds(g * 16, 16)]
                    for k in range(16):
                        e = g * 16 + k
                        a = a16[k]
                        for j in range(2):
                            dj = pl.ds(16 * j, 16)
                            m = s_rows[e, dj] + r_rows[e, dj] + t_rows[e, dj]
                            am = a * m
                            mw[e, dj] = am
                            mt[e, dj] = m - am
                    return gcarry

                lax.fori_loop(0, B // 16, group, 0)
                pltpu.sync_copy(mw, accw.at[objv], add=True)
                pltpu.sync_copy(mt, acct.at[objv], add=True)
                return carry

            lax.fori_loop(0, nblk_w, do_block, 0)
            plsc.subcore_barrier()
            pltpu.sync_copy(accw.at[sl], accw_o.at[q, sl])
            pltpu.sync_copy(acct.at[sl], acct_o.at[q, sl])

    return agg_kernel(hid4, rela4, ht4, alpha, sub4, rel4, tau4, obj, zrows)


def _final_tc(accw, acct, wh, whs, n):
    def body(aw_r, at_r, wh_r, whs_r, o1, o2):
        o1_acc = jnp.zeros((n, 128), F32)
        o2_acc = jnp.zeros((n, 128), F32)
        for q in range(4):
            o1_acc = o1_acc + jnp.dot(aw_r[q], wh_r[q], preferred_element_type=F32)
            o2_acc = o2_acc + jnp.dot(at_r[q], whs_r[q], preferred_element_type=F32)
        o1[...] = o1_acc
        o2[...] = o2_acc

    return pl.pallas_call(
        body,
        out_shape=(
            jax.ShapeDtypeStruct((n, 128), F32),
            jax.ShapeDtypeStruct((n, 128), F32),
        ),
    )(accw, acct, wh, whs)


def kernel(q_sub, q_rel, q_tau, hidden, edges, n_node, old_nodes_new_idx,
           rela_embed, Ws_attn, Wr_attn, Wqr_attn_w, Wqr_attn_b, Wtau_attn,
           w_alpha_w, w_alpha_b, W_h, W_h_s, weight_t1, bias_t1, weight_t2,
           bias_t2):
    n, d = hidden.shape
    nr = rela_embed.shape[0]

    ed = edges.astype(jnp.int32)
    sub = ed[:, 5]
    rel = ed[:, 2]
    obj = ed[:, 6]
    tau_raw = ed[:, 4]
    ridx = ed[:, 0]
    qt = jnp.asarray(q_tau, jnp.int32)
    tau = jnp.where(tau_raw >= 0, tau_raw, qt)

    delta = (jnp.arange(n, dtype=jnp.int32) - qt).astype(F32)[:, None]
    as_t, br_t, cf_t, dt_t, ht_t = _tables_tc(
        hidden, rela_embed, delta, Ws_attn, Wr_attn, Wqr_attn_w,
        Wqr_attn_b.reshape(1, 64), Wtau_attn, weight_t1, bias_t1,
        weight_t2, bias_t2)

    wa = w_alpha_w.reshape(64)
    wab = jnp.broadcast_to(w_alpha_b.reshape(1), (16,))
    alpha = _alpha_sc(as_t, br_t, cf_t, dt_t, sub, rel, tau, ridx,
                      q_rel.astype(jnp.int32), wa, wab)

    hid4 = jnp.concatenate([hidden[:, 32 * q:32 * (q + 1)] for q in range(4)],
                           axis=0)
    rela4 = jnp.concatenate(
        [rela_embed[:, 32 * q:32 * (q + 1)] for q in range(4)], axis=0)
    ht4 = jnp.concatenate([ht_t[:, 32 * q:32 * (q + 1)] for q in range(4)],
                          axis=0)
    sub4 = jnp.concatenate([sub + q * n for q in range(4)])
    rel4 = jnp.concatenate([rel + q * nr for q in range(4)])
    tau4 = jnp.concatenate([tau + q * n for q in range(4)])
    zrows = jnp.zeros((n // NS, 32), F32)
    accw, acct = _aggregate_sc(hid4, rela4, ht4, alpha, sub4, rel4, tau4,
                               obj, zrows, n)

    wh = W_h.reshape(4, 32, 128)
    whs = W_h_s.reshape(4, 32, 128)
    hidden_new, hidden_new_s = _final_tc(accw, acct, wh, whs, n)
    return (hidden_new, hidden_new_s)


# speedup vs baseline: 3.4265x; 1.2173x over previous
import functools

import jax
import jax.numpy as jnp
from jax import lax
from jax.experimental import pallas as pl
from jax.experimental.pallas import tpu as pltpu
from jax.experimental.pallas import tpu_sc as plsc

NC, NS, L = 2, 16, 16
B = 128
F32 = jnp.float32


def _tables_tc(hidden, rela, delta, wsa, wra, wqrw, wqrb, wtau, wt1, bt1, wt2, bt2):
    n = hidden.shape[0]
    nr = rela.shape[0]

    def body(h_r, r_r, d_r, wsa_r, wra_r, wqrw_r, wqrb_r, wtau_r,
             wt1_r, bt1_r, wt2_r, bt2_r, as_o, br_o, cf_o, dt_o, ht_o):
        d = d_r[...]
        ht = wt1_r[...] * d + bt1_r[...] + jnp.sin(wt2_r[...] * d + bt2_r[...])
        ht_o[...] = ht
        as_o[...] = jnp.dot(h_r[...], wsa_r[...], preferred_element_type=F32)
        br_o[...] = jnp.dot(r_r[...], wra_r[...], preferred_element_type=F32)
        cf_o[...] = jnp.dot(r_r[...], wqrw_r[...], preferred_element_type=F32) + wqrb_r[...]
        dt_o[...] = jnp.dot(ht, wtau_r[...], preferred_element_type=F32)

    return pl.pallas_call(
        body,
        out_shape=(
            jax.ShapeDtypeStruct((n, 64), F32),
            jax.ShapeDtypeStruct((nr, 64), F32),
            jax.ShapeDtypeStruct((nr, 64), F32),
            jax.ShapeDtypeStruct((n, 64), F32),
            jax.ShapeDtypeStruct((n, 128), F32),
        ),
        compiler_params=pltpu.CompilerParams(vmem_limit_bytes=100 * 1024 * 1024),
    )(hidden, rela, delta, wsa, wra, wqrw, wqrb, wtau, wt1, bt1, wt2, bt2)


def _alpha_sc(as_t, br_t, cf_t, dt_t, sub, rel, tau, ridx, qrel, wa, wab):
    ep = sub.shape[0]
    qp = qrel.shape[0]
    nblk = ep // B
    nw = NC * NS
    nblk_w = nblk // nw
    q_per_sub = qp // NS
    mesh = plsc.VectorSubcoreMesh(core_axis_name="c", subcore_axis_name="s",
                                  num_cores=NC, num_subcores=NS)

    @functools.partial(
        pl.kernel,
        out_type=jax.ShapeDtypeStruct((ep,), F32),
        mesh=mesh,
        scratch_types=[
            pltpu.VMEM((B,), jnp.int32),
            pltpu.VMEM((B,), jnp.int32),
            pltpu.VMEM((B,), jnp.int32),
            pltpu.VMEM((B,), jnp.int32),
            pltpu.VMEM((B,), jnp.int32),
            pltpu.VMEM((B, 64), F32),
            pltpu.VMEM((B, 64), F32),
            pltpu.VMEM((B, 64), F32),
            pltpu.VMEM((B, 64), F32),
            pltpu.VMEM((B,), F32),
            pltpu.VMEM((64,), F32),
            pltpu.VMEM((16,), F32),
            pltpu.SemaphoreType.DMA,
        ],
        compiler_params=pltpu.CompilerParams(needs_layout_passes=False,
                                             use_tc_tiling_on_sc=False),
    )
    def alpha_kernel(as_h, br_h, cf_h, dt_h, sub_h, rel_h, tau_h, ridx_h,
                     qrel_h, wa_h, wab_h, alpha_o,
                     subv, relv, tauv, ridxv, qv, s_rows, r_rows, c_rows,
                     d_rows, abuf, wav, wabv, sem):
        cid = lax.axis_index("c")
        sid = lax.axis_index("s")
        w = sid * NC + cid

        pltpu.sync_copy(wa_h, wav)
        pltpu.sync_copy(wab_h, wabv)
        wa_regs = [wav[pl.ds(16 * j, 16)] for j in range(4)]
        wab0 = wabv[...][0]
        lane = lax.iota(jnp.int32, 16)

        def do_block(i, carry):
            base = (w + i * nw) * B
            pltpu.sync_copy(sub_h.at[pl.ds(base, B)], subv)
            pltpu.sync_copy(rel_h.at[pl.ds(base, B)], relv)
            pltpu.sync_copy(tau_h.at[pl.ds(base, B)], tauv)
            pltpu.sync_copy(ridx_h.at[pl.ds(base, B)], ridxv)
            pltpu.async_copy(qrel_h.at[ridxv], qv, sem).wait()
            cp1 = pltpu.async_copy(as_h.at[subv], s_rows, sem)
            cp2 = pltpu.async_copy(br_h.at[relv], r_rows, sem)
            cp3 = pltpu.async_copy(dt_h.at[tauv], d_rows, sem)
            cp4 = pltpu.async_copy(cf_h.at[qv], c_rows, sem)
            cp1.wait(); cp2.wait(); cp3.wait(); cp4.wait()

            def group(g, gcarry):
                zv = jnp.zeros((16,), F32)
                for k in range(16):
                    e = g * 16 + k
                    vacc = None
                    for j in range(4):
                        dj = pl.ds(16 * j, 16)
                        y = (s_rows[e, dj] + r_rows[e, dj]
                             + c_rows[e, dj] + d_rows[e, dj])
                        y = jnp.maximum(y, 0.0) * wa_regs[j]
                        vacc = y if vacc is None else vacc + y
                    zv = jnp.where(lane == k, jnp.sum(vacc), zv)
                zv = zv + wab0
                abuf[pl.ds(g * 16, 16)] = 1.0 / (1.0 + jnp.exp(-zv))
                return gcarry

            lax.fori_loop(0, B // 16, group, 0)
            pltpu.sync_copy(abuf, alpha_o.at[pl.ds(base, B)])
            return carry

        lax.fori_loop(0, nblk_w, do_block, 0)

    return alpha_kernel(as_t, br_t, cf_t, dt_t, sub, rel, tau, ridx, qrel,
                        wa, wab)


def _aggregate_sc(hid4, rela4, ht4, alpha, sub4, rel4, tau4, obj, zrows, n, np_):
    ep = obj.shape[0]
    nblk = ep // B
    nblk_w = nblk // NS
    rows_per_sub = np_ // NS
    mesh = plsc.VectorSubcoreMesh(core_axis_name="c", subcore_axis_name="s",
                                  num_cores=NC, num_subcores=NS)

    @functools.partial(
        pl.kernel,
        out_type=(
            jax.ShapeDtypeStruct((4, np_, 32), F32),
            jax.ShapeDtypeStruct((4, np_, 32), F32),
        ),
        mesh=mesh,
        scratch_types=[
            pltpu.VMEM((B,), jnp.int32),
            pltpu.VMEM((B,), jnp.int32),
            pltpu.VMEM((B,), jnp.int32),
            pltpu.VMEM((B,), jnp.int32),
            pltpu.VMEM((B,), jnp.int32),
            pltpu.VMEM((B,), jnp.int32),
            pltpu.VMEM((B,), jnp.int32),
            pltpu.VMEM((B,), jnp.int32),
            pltpu.VMEM((B,), F32),
            pltpu.VMEM((B,), F32),
            pltpu.VMEM((B, 32), F32),
            pltpu.VMEM((B, 32), F32),
            pltpu.VMEM((B, 32), F32),
            pltpu.VMEM((B, 32), F32),
            pltpu.VMEM((B, 32), F32),
            pltpu.VMEM((B, 32), F32),
            pltpu.VMEM((B, 32), F32),
            pltpu.VMEM((B, 32), F32),
            pltpu.VMEM((rows_per_sub, 32), F32),
            pltpu.VMEM_SHARED((np_, 32), F32),
            pltpu.VMEM_SHARED((np_, 32), F32),
            pltpu.SemaphoreType.DMA,
            pltpu.SemaphoreType.DMA,
            pltpu.SemaphoreType.DMA,
            pltpu.SemaphoreType.DMA,
        ],
        compiler_params=pltpu.CompilerParams(needs_layout_passes=False,
                                             use_tc_tiling_on_sc=False),
    )
    def agg_kernel(hid_h, rela_h, ht_h, alpha_h, sub_h, rel_h, tau_h, obj_h,
                   z_h, accw_o, acct_o,
                   subv0, subv1, relv0, relv1, tauv0, tauv1, objv0, objv1,
                   av0, av1, s_rows0, s_rows1, r_rows0, r_rows1,
                   t_rows0, t_rows1, mw, mt, zv, accw, acct,
                   semI0, semI1, semR0, semR1):
        cid = lax.axis_index("c")
        sid = lax.axis_index("s")
        subv = (subv0, subv1)
        relv = (relv0, relv1)
        tauv = (tauv0, tauv1)
        objv = (objv0, objv1)
        av = (av0, av1)
        s_rows = (s_rows0, s_rows1)
        r_rows = (r_rows0, r_rows1)
        t_rows = (t_rows0, t_rows1)
        semI = (semI0, semI1)
        semR = (semR0, semR1)

        pltpu.sync_copy(z_h, zv)
        sl = pl.ds(sid * rows_per_sub, rows_per_sub)

        for ph in range(2):
            qq = cid * 2 + ph

            pltpu.sync_copy(zv, accw.at[sl])
            pltpu.sync_copy(zv, acct.at[sl])
            plsc.subcore_barrier()

            def issue_idx(i, p):
                base = (sid + i * NS) * B
                ebase = qq * ep + base
                pltpu.async_copy(sub_h.at[pl.ds(ebase, B)], subv[p], semI[p])
                pltpu.async_copy(rel_h.at[pl.ds(ebase, B)], relv[p], semI[p])
                pltpu.async_copy(tau_h.at[pl.ds(ebase, B)], tauv[p], semI[p])
                pltpu.async_copy(obj_h.at[pl.ds(base, B)], objv[p], semI[p])
                pltpu.async_copy(alpha_h.at[pl.ds(base, B)], av[p], semI[p])

            def wait_idx(p):
                pltpu.make_async_copy(sub_h.at[pl.ds(0, B)], subv[p], semI[p]).wait()
                pltpu.make_async_copy(rel_h.at[pl.ds(0, B)], relv[p], semI[p]).wait()
                pltpu.make_async_copy(tau_h.at[pl.ds(0, B)], tauv[p], semI[p]).wait()
                pltpu.make_async_copy(obj_h.at[pl.ds(0, B)], objv[p], semI[p]).wait()
                pltpu.make_async_copy(alpha_h.at[pl.ds(0, B)], av[p], semI[p]).wait()

            def issue_rows(p):
                pltpu.async_copy(hid_h.at[subv[p]], s_rows[p], semR[p])
                pltpu.async_copy(rela_h.at[relv[p]], r_rows[p], semR[p])
                pltpu.async_copy(ht_h.at[tauv[p]], t_rows[p], semR[p])

            def wait_rows(p):
                pltpu.make_async_copy(hid_h.at[subv[p]], s_rows[p], semR[p]).wait()
                pltpu.make_async_copy(rela_h.at[relv[p]], r_rows[p], semR[p]).wait()
                pltpu.make_async_copy(ht_h.at[tauv[p]], t_rows[p], semR[p]).wait()

            def compute(i, p):
                sr, rr, tr = s_rows[p], r_rows[p], t_rows[p]
                avp = av[p]

                def group(g, gcarry):
                    a16 = avp[pl.ds(g * 16, 16)]
                    for k in range(16):
                        e = g * 16 + k
                        a = a16[k]
                        for j in range(2):
                            dj = pl.ds(16 * j, 16)
                            m = sr[e, dj] + rr[e, dj] + tr[e, dj]
                            am = a * m
                            mw[e, dj] = am
                            mt[e, dj] = m - am
                    return gcarry

                lax.fori_loop(0, B // 16, group, 0)
                pltpu.sync_copy(mw, accw.at[objv[p]], add=True)
                pltpu.sync_copy(mt, acct.at[objv[p]], add=True)

            def pipe_body(i, p):
                q = 1 - p

                @pl.when(i + 1 < nblk_w)
                def _():
                    wait_idx(q)
                    issue_rows(q)

                wait_rows(p)
                compute(i, p)

                @pl.when(i + 2 < nblk_w)
                def _():
                    issue_idx(i + 2, p)

            issue_idx(0, 0)
            issue_idx(1, 1)
            wait_idx(0)
            issue_rows(0)

            def pstep(t, carry):
                pipe_body(t * 2, 0)
                pipe_body(t * 2 + 1, 1)
                return carry

            lax.fori_loop(0, nblk_w // 2, pstep, 0)

            plsc.subcore_barrier()
            pltpu.sync_copy(accw.at[sl], accw_o.at[qq, sl])
            pltpu.sync_copy(acct.at[sl], acct_o.at[qq, sl])

    return agg_kernel(hid4, rela4, ht4, alpha, sub4, rel4, tau4, obj, zrows)


def _final_tc(accw, acct, wh, whs, n, np_):
    def body(aw_r, at_r, wh_r, whs_r, o1, o2):
        o1_acc = jnp.zeros((np_, 128), F32)
        o2_acc = jnp.zeros((np_, 128), F32)
        for q in range(4):
            o1_acc = o1_acc + jnp.dot(aw_r[q], wh_r[q], preferred_element_type=F32)
            o2_acc = o2_acc + jnp.dot(at_r[q], whs_r[q], preferred_element_type=F32)
        o1[...] = o1_acc[:n]
        o2[...] = o2_acc[:n]

    return pl.pallas_call(
        body,
        out_shape=(
            jax.ShapeDtypeStruct((n, 128), F32),
            jax.ShapeDtypeStruct((n, 128), F32),
        ),
        compiler_params=pltpu.CompilerParams(vmem_limit_bytes=100 * 1024 * 1024),
    )(accw, acct, wh, whs)


def kernel(q_sub, q_rel, q_tau, hidden, edges, n_node, old_nodes_new_idx,
           rela_embed, Ws_attn, Wr_attn, Wqr_attn_w, Wqr_attn_b, Wtau_attn,
           w_alpha_w, w_alpha_b, W_h, W_h_s, weight_t1, bias_t1, weight_t2,
           bias_t2):
    n, d = hidden.shape
    nr = rela_embed.shape[0]
    e = edges.shape[0]
    nwk = NC * NS
    blk_unit = 2 * B * nwk
    ep = ((e + blk_unit - 1) // blk_unit) * blk_unit
    np_ = ((n + NS - 1) // NS) * NS + NS
    qp = ((q_rel.shape[0] + B * NS - 1) // (B * NS)) * (B * NS)

    ed = edges.astype(jnp.int32)
    qt = jnp.asarray(q_tau, jnp.int32)

    def padi(x, value):
        return jnp.concatenate(
            [x, jnp.full((ep - e,), value, jnp.int32)]) if ep > e else x

    sub = padi(ed[:, 5], 0)
    rel = padi(ed[:, 2], 0)
    obj = padi(ed[:, 6], n)
    ridx = padi(ed[:, 0], 0)
    tau_raw = ed[:, 4]
    tau = padi(jnp.where(tau_raw >= 0, tau_raw, qt), 0)
    qrel = jnp.concatenate(
        [q_rel.astype(jnp.int32),
         jnp.zeros((qp - q_rel.shape[0],), jnp.int32)])

    delta = (jnp.arange(n, dtype=jnp.int32) - qt).astype(F32)[:, None]
    as_t, br_t, cf_t, dt_t, ht_t = _tables_tc(
        hidden, rela_embed, delta, Ws_attn, Wr_attn, Wqr_attn_w,
        Wqr_attn_b.reshape(1, 64), Wtau_attn, weight_t1, bias_t1,
        weight_t2, bias_t2)

    wa = w_alpha_w.reshape(64)
    wab = jnp.broadcast_to(w_alpha_b.reshape(1), (16,))
    alpha = _alpha_sc(as_t, br_t, cf_t, dt_t, sub, rel, tau, ridx, qrel,
                      wa, wab)

    hid4 = jnp.concatenate([hidden[:, 32 * q:32 * (q + 1)] for q in range(4)],
                           axis=0)
    rela4 = jnp.concatenate(
        [rela_embed[:, 32 * q:32 * (q + 1)] for q in range(4)], axis=0)
    ht4 = jnp.concatenate([ht_t[:, 32 * q:32 * (q + 1)] for q in range(4)],
                          axis=0)
    sub4 = jnp.concatenate([sub + q * n for q in range(4)])
    rel4 = jnp.concatenate([rel + q * nr for q in range(4)])
    tau4 = jnp.concatenate([tau + q * n for q in range(4)])
    zrows = jnp.zeros((np_ // NS, 32), F32)
    accw, acct = _aggregate_sc(hid4, rela4, ht4, alpha, sub4, rel4, tau4,
                               obj, zrows, n, np_)

    wh = W_h.reshape(4, 32, 128)
    whs = W_h_s.reshape(4, 32, 128)
    hidden_new, hidden_new_s = _final_tc(accw, acct, wh, whs, n, np_)
    return (hidden_new, hidden_new_s)


# speedup vs baseline: 4.0588x; 1.1845x over previous
import functools

import jax
import jax.numpy as jnp
from jax import lax
from jax.experimental import pallas as pl
from jax.experimental.pallas import tpu as pltpu
from jax.experimental.pallas import tpu_sc as plsc

NC, NS, L = 2, 16, 16
B = 128
F32 = jnp.float32


_HP = lax.Precision.HIGHEST


def _tables_tc(hidden, rela, delta, wsa, wra, wqrw, wqrb, wtau, wt1, bt1, wt2, bt2):
    n = hidden.shape[0]
    nr = rela.shape[0]

    def body1(h_r, r_r, wsa_r, wra_r, wqrw_r, wqrb_r, as_o, br_o, cf_o):
        as_o[...] = jnp.dot(h_r[...], wsa_r[...], preferred_element_type=F32,
                            precision=_HP)
        br_o[...] = jnp.dot(r_r[...], wra_r[...], preferred_element_type=F32,
                            precision=_HP)
        cf_o[...] = jnp.dot(r_r[...], wqrw_r[...], preferred_element_type=F32,
                            precision=_HP) + wqrb_r[...]

    as_t, br_t, cf_t = pl.pallas_call(
        body1,
        out_shape=(
            jax.ShapeDtypeStruct((n, 64), F32),
            jax.ShapeDtypeStruct((nr, 64), F32),
            jax.ShapeDtypeStruct((nr, 64), F32),
        ),
        compiler_params=pltpu.CompilerParams(vmem_limit_bytes=60 * 1024 * 1024),
    )(hidden, rela, wsa, wra, wqrw, wqrb)

    def body2(d_r, wtau_r, wt1_r, bt1_r, wt2_r, bt2_r, ht_o, dt_o):
        d = d_r[...]
        ht = wt1_r[...] * d + bt1_r[...] + jnp.sin(wt2_r[...] * d + bt2_r[...])
        ht_o[...] = ht
        dt_o[...] = jnp.dot(ht, wtau_r[...], preferred_element_type=F32,
                            precision=_HP)

    ht_t, dt_t = pl.pallas_call(
        body2,
        out_shape=(
            jax.ShapeDtypeStruct((n, 128), F32),
            jax.ShapeDtypeStruct((n, 64), F32),
        ),
        compiler_params=pltpu.CompilerParams(vmem_limit_bytes=60 * 1024 * 1024),
    )(delta, wtau, wt1, bt1, wt2, bt2)

    return as_t, br_t, cf_t, dt_t, ht_t


def _sigmoid_tc(z2d):
    def body(z_r, a_o):
        a_o[...] = jax.nn.sigmoid(z_r[...])

    return pl.pallas_call(
        body,
        out_shape=jax.ShapeDtypeStruct(z2d.shape, F32),
    )(z2d)


def _cq_sc(cf_t, qrel):
    qp = qrel.shape[0]
    nblk0 = qp // B
    nw = NC * NS
    mesh = plsc.VectorSubcoreMesh(core_axis_name="c", subcore_axis_name="s",
                                  num_cores=NC, num_subcores=NS)

    @functools.partial(
        pl.kernel,
        out_type=jax.ShapeDtypeStruct((qp, 64), F32),
        mesh=mesh,
        scratch_types=[
            pltpu.VMEM((B,), jnp.int32),
            pltpu.VMEM((B, 64), F32),
            pltpu.SemaphoreType.DMA,
        ],
        compiler_params=pltpu.CompilerParams(needs_layout_passes=False,
                                             use_tc_tiling_on_sc=False),
    )
    def cq_kernel(cf_h, qrel_h, cq_o, qiv, rows, sem):
        cid = lax.axis_index("c")
        sid = lax.axis_index("s")
        w = sid * NC + cid
        nblk_w = (nblk0 - w + nw - 1) // nw

        def do_block(i, carry):
            base = (w + i * nw) * B
            pltpu.sync_copy(qrel_h.at[pl.ds(base, B)], qiv)
            pltpu.async_copy(cf_h.at[qiv], rows, sem).wait()
            pltpu.sync_copy(rows, cq_o.at[pl.ds(base, B)])
            return carry

        lax.fori_loop(0, nblk_w, do_block, 0)

    return cq_kernel(cf_t, qrel)


def _alpha_sc(as_t, br_t, cq_t, dt_t, sub, rel, tau, ridx, wa, wab):
    ep = sub.shape[0]
    nblk = ep // B
    nw = NC * NS
    nblk_w = nblk // nw
    mesh = plsc.VectorSubcoreMesh(core_axis_name="c", subcore_axis_name="s",
                                  num_cores=NC, num_subcores=NS)

    @functools.partial(
        pl.kernel,
        out_type=jax.ShapeDtypeStruct((ep,), F32),
        mesh=mesh,
        scratch_types=[
            pltpu.VMEM((B,), jnp.int32),
            pltpu.VMEM((B,), jnp.int32),
            pltpu.VMEM((B,), jnp.int32),
            pltpu.VMEM((B,), jnp.int32),
            pltpu.VMEM((B,), jnp.int32),
            pltpu.VMEM((B,), jnp.int32),
            pltpu.VMEM((B,), jnp.int32),
            pltpu.VMEM((B,), jnp.int32),
            pltpu.VMEM((B, 64), F32),
            pltpu.VMEM((B, 64), F32),
            pltpu.VMEM((B, 64), F32),
            pltpu.VMEM((B, 64), F32),
            pltpu.VMEM((B, 64), F32),
            pltpu.VMEM((B, 64), F32),
            pltpu.VMEM((B, 64), F32),
            pltpu.VMEM((B, 64), F32),
            pltpu.VMEM((B,), F32),
            pltpu.VMEM((64,), F32),
            pltpu.VMEM((16,), F32),
            pltpu.SemaphoreType.DMA,
            pltpu.SemaphoreType.DMA,
            pltpu.SemaphoreType.DMA,
            pltpu.SemaphoreType.DMA,
        ],
        compiler_params=pltpu.CompilerParams(needs_layout_passes=False,
                                             use_tc_tiling_on_sc=False),
    )
    def alpha_kernel(as_h, br_h, cq_h, dt_h, sub_h, rel_h, tau_h, ridx_h,
                     wa_h, wab_h, alpha_o,
                     subv0, subv1, relv0, relv1, tauv0, tauv1, ridxv0, ridxv1,
                     s_rows0, s_rows1, r_rows0, r_rows1, c_rows0, c_rows1,
                     d_rows0, d_rows1, abuf, wav, wabv,
                     semI0, semI1, semR0, semR1):
        cid = lax.axis_index("c")
        sid = lax.axis_index("s")
        w = sid * NC + cid
        subv = (subv0, subv1)
        relv = (relv0, relv1)
        tauv = (tauv0, tauv1)
        ridxv = (ridxv0, ridxv1)
        s_rows = (s_rows0, s_rows1)
        r_rows = (r_rows0, r_rows1)
        c_rows = (c_rows0, c_rows1)
        d_rows = (d_rows0, d_rows1)
        semI = (semI0, semI1)
        semR = (semR0, semR1)

        pltpu.sync_copy(wa_h, wav)
        pltpu.sync_copy(wab_h, wabv)
        wa_regs = [wav[pl.ds(16 * j, 16)] for j in range(4)]
        wab0 = wabv[...][0]
        lane = lax.iota(jnp.int32, 16)

        idx_groups = ((sub_h, subv), (rel_h, relv), (tau_h, tauv),
                      (ridx_h, ridxv))

        def issue_idx(i, p):
            base = (w + i * nw) * B
            for hbm, bufs in idx_groups:
                pltpu.async_copy(hbm.at[pl.ds(base, B)], bufs[p], semI[p])

        def wait_idx(p):
            for hbm, bufs in idx_groups:
                pltpu.make_async_copy(hbm.at[pl.ds(0, B)], bufs[p],
                                      semI[p]).wait()

        def issue_rows(p):
            pltpu.async_copy(as_h.at[subv[p]], s_rows[p], semR[p])
            pltpu.async_copy(br_h.at[relv[p]], r_rows[p], semR[p])
            pltpu.async_copy(dt_h.at[tauv[p]], d_rows[p], semR[p])
            pltpu.async_copy(cq_h.at[ridxv[p]], c_rows[p], semR[p])

        def wait_rows(p):
            pltpu.make_async_copy(as_h.at[subv[p]], s_rows[p], semR[p]).wait()
            pltpu.make_async_copy(br_h.at[relv[p]], r_rows[p], semR[p]).wait()
            pltpu.make_async_copy(dt_h.at[tauv[p]], d_rows[p], semR[p]).wait()
            pltpu.make_async_copy(cq_h.at[ridxv[p]], c_rows[p], semR[p]).wait()

        def compute(i, p):
            base = (w + i * nw) * B
            sr, rr, cr, dr = s_rows[p], r_rows[p], c_rows[p], d_rows[p]

            def group(g, gcarry):
                zv = jnp.zeros((16,), F32)
                for k in range(16):
                    e = g * 16 + k
                    vacc = None
                    for j in range(4):
                        dj = pl.ds(16 * j, 16)
                        y = sr[e, dj] + rr[e, dj] + cr[e, dj] + dr[e, dj]
                        y = jnp.maximum(y, 0.0) * wa_regs[j]
                        vacc = y if vacc is None else vacc + y
                    zv = jnp.where(lane == k, jnp.sum(vacc), zv)
                abuf[pl.ds(g * 16, 16)] = zv + wab0
                return gcarry

            lax.fori_loop(0, B // 16, group, 0)
            pltpu.sync_copy(abuf, alpha_o.at[pl.ds(base, B)])

        def pipe_body(i, p):
            q = 1 - p

            @pl.when(i + 1 < nblk_w)
            def _():
                wait_idx(q)
                issue_rows(q)

            wait_rows(p)
            compute(i, p)

            @pl.when(i + 2 < nblk_w)
            def _():
                issue_idx(i + 2, p)

        issue_idx(0, 0)
        issue_idx(1, 1)
        wait_idx(0)
        issue_rows(0)

        def step(t, carry):
            pipe_body(t * 2, 0)
            pipe_body(t * 2 + 1, 1)
            return carry

        lax.fori_loop(0, nblk_w // 2, step, 0)

    return alpha_kernel(as_t, br_t, cq_t, dt_t, sub, rel, tau, ridx, wa, wab)


def _aggregate_sc(hid4, rela4, ht4, alpha, sub4, rel4, tau4, obj, zrows, n, np_):
    ep = obj.shape[0]
    nblk = ep // B
    nblk_w = nblk // NS
    rows_per_sub = np_ // NS
    mesh = plsc.VectorSubcoreMesh(core_axis_name="c", subcore_axis_name="s",
                                  num_cores=NC, num_subcores=NS)

    @functools.partial(
        pl.kernel,
        out_type=(
            jax.ShapeDtypeStruct((4, np_, 32), F32),
            jax.ShapeDtypeStruct((4, np_, 32), F32),
        ),
        mesh=mesh,
        scratch_types=[
            pltpu.VMEM((B,), jnp.int32),
            pltpu.VMEM((B,), jnp.int32),
            pltpu.VMEM((B,), jnp.int32),
            pltpu.VMEM((B,), jnp.int32),
            pltpu.VMEM((B,), jnp.int32),
            pltpu.VMEM((B,), jnp.int32),
            pltpu.VMEM((B,), jnp.int32),
            pltpu.VMEM((B,), jnp.int32),
            pltpu.VMEM((B,), F32),
            pltpu.VMEM((B,), F32),
            pltpu.VMEM((B, 32), F32),
            pltpu.VMEM((B, 32), F32),
            pltpu.VMEM((B, 32), F32),
            pltpu.VMEM((B, 32), F32),
            pltpu.VMEM((B, 32), F32),
            pltpu.VMEM((B, 32), F32),
            pltpu.VMEM((B, 32), F32),
            pltpu.VMEM((B, 32), F32),
            pltpu.VMEM((rows_per_sub, 32), F32),
            pltpu.VMEM_SHARED((np_, 32), F32),
            pltpu.VMEM_SHARED((np_, 32), F32),
            pltpu.SemaphoreType.DMA,
            pltpu.SemaphoreType.DMA,
            pltpu.SemaphoreType.DMA,
            pltpu.SemaphoreType.DMA,
        ],
        compiler_params=pltpu.CompilerParams(needs_layout_passes=False,
                                             use_tc_tiling_on_sc=False),
    )
    def agg_kernel(hid_h, rela_h, ht_h, alpha_h, sub_h, rel_h, tau_h, obj_h,
                   z_h, accw_o, acct_o,
                   subv0, subv1, relv0, relv1, tauv0, tauv1, objv0, objv1,
                   av0, av1, s_rows0, s_rows1, r_rows0, r_rows1,
                   t_rows0, t_rows1, mw, mt, zv, accw, acct,
                   semI0, semI1, semR0, semR1):
        cid = lax.axis_index("c")
        sid = lax.axis_index("s")
        subv = (subv0, subv1)
        relv = (relv0, relv1)
        tauv = (tauv0, tauv1)
        objv = (objv0, objv1)
        av = (av0, av1)
        s_rows = (s_rows0, s_rows1)
        r_rows = (r_rows0, r_rows1)
        t_rows = (t_rows0, t_rows1)
        semI = (semI0, semI1)
        semR = (semR0, semR1)

        pltpu.sync_copy(z_h, zv)
        sl = pl.ds(sid * rows_per_sub, rows_per_sub)

        for ph in range(2):
            qq = cid * 2 + ph

            pltpu.sync_copy(zv, accw.at[sl])
            pltpu.sync_copy(zv, acct.at[sl])
            plsc.subcore_barrier()

            def issue_idx(i, p):
                base = (sid + i * NS) * B
                ebase = qq * ep + base
                pltpu.async_copy(sub_h.at[pl.ds(ebase, B)], subv[p], semI[p])
                pltpu.async_copy(rel_h.at[pl.ds(ebase, B)], relv[p], semI[p])
                pltpu.async_copy(tau_h.at[pl.ds(ebase, B)], tauv[p], semI[p])
                pltpu.async_copy(obj_h.at[pl.ds(base, B)], objv[p], semI[p])
                pltpu.async_copy(alpha_h.at[pl.ds(base, B)], av[p], semI[p])

            def wait_idx(p):
                pltpu.make_async_copy(sub_h.at[pl.ds(0, B)], subv[p], semI[p]).wait()
                pltpu.make_async_copy(rel_h.at[pl.ds(0, B)], relv[p], semI[p]).wait()
                pltpu.make_async_copy(tau_h.at[pl.ds(0, B)], tauv[p], semI[p]).wait()
                pltpu.make_async_copy(obj_h.at[pl.ds(0, B)], objv[p], semI[p]).wait()
                pltpu.make_async_copy(alpha_h.at[pl.ds(0, B)], av[p], semI[p]).wait()

            def issue_rows(p):
                pltpu.async_copy(hid_h.at[subv[p]], s_rows[p], semR[p])
                pltpu.async_copy(rela_h.at[relv[p]], r_rows[p], semR[p])
                pltpu.async_copy(ht_h.at[tauv[p]], t_rows[p], semR[p])

            def wait_rows(p):
                pltpu.make_async_copy(hid_h.at[subv[p]], s_rows[p], semR[p]).wait()
                pltpu.make_async_copy(rela_h.at[relv[p]], r_rows[p], semR[p]).wait()
                pltpu.make_async_copy(ht_h.at[tauv[p]], t_rows[p], semR[p]).wait()

            def compute(i, p):
                sr, rr, tr = s_rows[p], r_rows[p], t_rows[p]
                avp = av[p]

                def group(g, gcarry):
                    a16 = avp[pl.ds(g * 16, 16)]
                    for k in range(16):
                        e = g * 16 + k
                        a = a16[k]
                        for j in range(2):
                            dj = pl.ds(16 * j, 16)
                            m = sr[e, dj] + rr[e, dj] + tr[e, dj]
                            am = a * m
                            mw[e, dj] = am
                            mt[e, dj] = m - am
                    return gcarry

                lax.fori_loop(0, B // 16, group, 0)
                pltpu.sync_copy(mw, accw.at[objv[p]], add=True)
                pltpu.sync_copy(mt, acct.at[objv[p]], add=True)

            def pipe_body(i, p):
                q = 1 - p

                @pl.when(i + 1 < nblk_w)
                def _():
                    wait_idx(q)
                    issue_rows(q)

                wait_rows(p)
                compute(i, p)

                @pl.when(i + 2 < nblk_w)
                def _():
                    issue_idx(i + 2, p)

            issue_idx(0, 0)
            issue_idx(1, 1)
            wait_idx(0)
            issue_rows(0)

            def pstep(t, carry):
                pipe_body(t * 2, 0)
                pipe_body(t * 2 + 1, 1)
                return carry

            lax.fori_loop(0, nblk_w // 2, pstep, 0)

            plsc.subcore_barrier()
            pltpu.sync_copy(accw.at[sl], accw_o.at[qq, sl])
            pltpu.sync_copy(acct.at[sl], acct_o.at[qq, sl])

    return agg_kernel(hid4, rela4, ht4, alpha, sub4, rel4, tau4, obj, zrows)


def _final_tc(accw, acct, wh, whs, n, np_):
    def body(aw_r, at_r, wh_r, whs_r, o1, o2):
        o1[...] = jnp.dot(aw_r[...], wh_r[...], preferred_element_type=F32,
                          precision=_HP)[:n]
        o2[...] = jnp.dot(at_r[...], whs_r[...], preferred_element_type=F32,
                          precision=_HP)[:n]

    return pl.pallas_call(
        body,
        out_shape=(
            jax.ShapeDtypeStruct((n, 128), F32),
            jax.ShapeDtypeStruct((n, 128), F32),
        ),
        compiler_params=pltpu.CompilerParams(vmem_limit_bytes=100 * 1024 * 1024),
    )(accw, acct, wh, whs)


def kernel(q_sub, q_rel, q_tau, hidden, edges, n_node, old_nodes_new_idx,
           rela_embed, Ws_attn, Wr_attn, Wqr_attn_w, Wqr_attn_b, Wtau_attn,
           w_alpha_w, w_alpha_b, W_h, W_h_s, weight_t1, bias_t1, weight_t2,
           bias_t2):
    n, d = hidden.shape
    nr = rela_embed.shape[0]
    e = edges.shape[0]
    nwk = NC * NS
    blk_unit = 2 * B * nwk
    ep = ((e + blk_unit - 1) // blk_unit) * blk_unit
    np_ = ((n + NS - 1) // NS) * NS + NS
    qp = ((q_rel.shape[0] + B * NS - 1) // (B * NS)) * (B * NS)

    ed = edges.astype(jnp.int32)
    qt = jnp.asarray(q_tau, jnp.int32)

    def padi(x, value):
        return jnp.concatenate(
            [x, jnp.full((ep - e,), value, jnp.int32)]) if ep > e else x

    sub = padi(ed[:, 5], 0)
    rel = padi(ed[:, 2], 0)
    obj = padi(ed[:, 6], n)
    ridx = padi(ed[:, 0], 0)
    tau_raw = ed[:, 4]
    tau = padi(jnp.where(tau_raw >= 0, tau_raw, qt), 0)
    qrel = jnp.concatenate(
        [q_rel.astype(jnp.int32),
         jnp.zeros((qp - q_rel.shape[0],), jnp.int32)])

    delta = (jnp.arange(n, dtype=jnp.int32) - qt).astype(F32)[:, None]
    as_t, br_t, cf_t, dt_t, ht_t = _tables_tc(
        hidden, rela_embed, delta, Ws_attn, Wr_attn, Wqr_attn_w,
        Wqr_attn_b.reshape(1, 64), Wtau_attn, weight_t1, bias_t1,
        weight_t2, bias_t2)

    cq_t = _cq_sc(cf_t, qrel)
    wa = w_alpha_w.reshape(64)
    wab = jnp.broadcast_to(w_alpha_b.reshape(1), (16,))
    zlogit = _alpha_sc(as_t, br_t, cq_t, dt_t, sub, rel, tau, ridx, wa, wab)
    alpha = _sigmoid_tc(zlogit.reshape(ep // 128, 128)).reshape(ep)

    hid4 = jnp.concatenate([hidden[:, 32 * q:32 * (q + 1)] for q in range(4)],
                           axis=0)
    rela4 = jnp.concatenate(
        [rela_embed[:, 32 * q:32 * (q + 1)] for q in range(4)], axis=0)
    ht4 = jnp.concatenate([ht_t[:, 32 * q:32 * (q + 1)] for q in range(4)],
                          axis=0)
    sub4 = jnp.concatenate([sub + q * n for q in range(4)])
    rel4 = jnp.concatenate([rel + q * nr for q in range(4)])
    tau4 = jnp.concatenate([tau + q * n for q in range(4)])
    zrows = jnp.zeros((np_ // NS, 32), F32)
    accw, acct = _aggregate_sc(hid4, rela4, ht4, alpha, sub4, rel4, tau4,
                               obj, zrows, n, np_)

    accw_f = jnp.moveaxis(accw, 0, 1).reshape(np_, 128)
    acct_f = jnp.moveaxis(acct, 0, 1).reshape(np_, 128)
    hidden_new, hidden_new_s = _final_tc(accw_f, acct_f, W_h, W_h_s, n, np_)
    return (hidden_new, hidden_new_s)


# speedup vs baseline: 4.4034x; 1.0849x over previous
import functools

import jax
import jax.numpy as jnp
from jax import lax
from jax.experimental import pallas as pl
from jax.experimental.pallas import tpu as pltpu
from jax.experimental.pallas import tpu_sc as plsc

NC, NS, L = 2, 16, 16
B = 128
F32 = jnp.float32


_HP = lax.Precision.HIGHEST


def _tables_tc(hidden, rela, delta, wsa, wra, wqrw, wqrb, wtau, wt1, bt1, wt2, bt2):
    n = hidden.shape[0]
    nr = rela.shape[0]

    def body1(h_r, r_r, wsa_r, wra_r, wqrw_r, wqrb_r, as_o, br_o, cf_o):
        as_o[...] = jnp.dot(h_r[...], wsa_r[...], preferred_element_type=F32,
                            precision=_HP)
        br_o[...] = jnp.dot(r_r[...], wra_r[...], preferred_element_type=F32,
                            precision=_HP)
        cf_o[...] = jnp.dot(r_r[...], wqrw_r[...], preferred_element_type=F32,
                            precision=_HP) + wqrb_r[...]

    as_t, br_t, cf_t = pl.pallas_call(
        body1,
        out_shape=(
            jax.ShapeDtypeStruct((n, 64), F32),
            jax.ShapeDtypeStruct((nr, 64), F32),
            jax.ShapeDtypeStruct((nr, 64), F32),
        ),
        compiler_params=pltpu.CompilerParams(vmem_limit_bytes=60 * 1024 * 1024),
    )(hidden, rela, wsa, wra, wqrw, wqrb)

    def body2(d_r, wtau_r, wt1_r, bt1_r, wt2_r, bt2_r, ht_o, dt_o):
        d = d_r[...]
        ht = wt1_r[...] * d + bt1_r[...] + jnp.sin(wt2_r[...] * d + bt2_r[...])
        ht_o[...] = ht
        dt_o[...] = jnp.dot(ht, wtau_r[...], preferred_element_type=F32,
                            precision=_HP)

    ht_t, dt_t = pl.pallas_call(
        body2,
        out_shape=(
            jax.ShapeDtypeStruct((n, 128), F32),
            jax.ShapeDtypeStruct((n, 64), F32),
        ),
        compiler_params=pltpu.CompilerParams(vmem_limit_bytes=60 * 1024 * 1024),
    )(delta, wtau, wt1, bt1, wt2, bt2)

    return as_t, br_t, cf_t, dt_t, ht_t


def _sigmoid_tc(z2d):
    def body(z_r, a_o):
        a_o[...] = jax.nn.sigmoid(z_r[...])

    return pl.pallas_call(
        body,
        out_shape=jax.ShapeDtypeStruct(z2d.shape, F32),
    )(z2d)


def _cq_sc(cf_t, qrel):
    qp = qrel.shape[0]
    nblk0 = qp // B
    nw = NC * NS
    mesh = plsc.VectorSubcoreMesh(core_axis_name="c", subcore_axis_name="s",
                                  num_cores=NC, num_subcores=NS)

    @functools.partial(
        pl.kernel,
        out_type=jax.ShapeDtypeStruct((qp, 64), F32),
        mesh=mesh,
        scratch_types=[
            pltpu.VMEM((B,), jnp.int32),
            pltpu.VMEM((B, 64), F32),
            pltpu.SemaphoreType.DMA,
        ],
        compiler_params=pltpu.CompilerParams(needs_layout_passes=False,
                                             use_tc_tiling_on_sc=False),
    )
    def cq_kernel(cf_h, qrel_h, cq_o, qiv, rows, sem):
        cid = lax.axis_index("c")
        sid = lax.axis_index("s")
        w = sid * NC + cid
        nblk_w = (nblk0 - w + nw - 1) // nw

        def do_block(i, carry):
            base = (w + i * nw) * B
            pltpu.sync_copy(qrel_h.at[pl.ds(base, B)], qiv)
            pltpu.async_copy(cf_h.at[qiv], rows, sem).wait()
            pltpu.sync_copy(rows, cq_o.at[pl.ds(base, B)])
            return carry

        lax.fori_loop(0, nblk_w, do_block, 0)

    return cq_kernel(cf_t, qrel)


def _alpha_sc(as_t, br_t, cq_t, dt_t, sub, rel, tau, ridx, wa, wab):
    ep = sub.shape[0]
    nblk = ep // B
    nw = NC * NS
    nblk_w = nblk // nw
    mesh = plsc.VectorSubcoreMesh(core_axis_name="c", subcore_axis_name="s",
                                  num_cores=NC, num_subcores=NS)

    @functools.partial(
        pl.kernel,
        out_type=jax.ShapeDtypeStruct((ep,), F32),
        mesh=mesh,
        scratch_types=[
            pltpu.VMEM((B,), jnp.int32),
            pltpu.VMEM((B,), jnp.int32),
            pltpu.VMEM((B,), jnp.int32),
            pltpu.VMEM((B,), jnp.int32),
            pltpu.VMEM((B,), jnp.int32),
            pltpu.VMEM((B,), jnp.int32),
            pltpu.VMEM((B,), jnp.int32),
            pltpu.VMEM((B,), jnp.int32),
            pltpu.VMEM((B, 64), F32),
            pltpu.VMEM((B, 64), F32),
            pltpu.VMEM((B, 64), F32),
            pltpu.VMEM((B, 64), F32),
            pltpu.VMEM((B, 64), F32),
            pltpu.VMEM((B, 64), F32),
            pltpu.VMEM((B, 64), F32),
            pltpu.VMEM((B, 64), F32),
            pltpu.VMEM((B,), F32),
            pltpu.VMEM((64,), F32),
            pltpu.VMEM((16,), F32),
            pltpu.SemaphoreType.DMA,
            pltpu.SemaphoreType.DMA,
            pltpu.SemaphoreType.DMA,
            pltpu.SemaphoreType.DMA,
        ],
        compiler_params=pltpu.CompilerParams(needs_layout_passes=False,
                                             use_tc_tiling_on_sc=False),
    )
    def alpha_kernel(as_h, br_h, cq_h, dt_h, sub_h, rel_h, tau_h, ridx_h,
                     wa_h, wab_h, alpha_o,
                     subv0, subv1, relv0, relv1, tauv0, tauv1, ridxv0, ridxv1,
                     s_rows0, s_rows1, r_rows0, r_rows1, c_rows0, c_rows1,
                     d_rows0, d_rows1, abuf, wav, wabv,
                     semI0, semI1, semR0, semR1):
        cid = lax.axis_index("c")
        sid = lax.axis_index("s")
        w = sid * NC + cid
        subv = (subv0, subv1)
        relv = (relv0, relv1)
        tauv = (tauv0, tauv1)
        ridxv = (ridxv0, ridxv1)
        s_rows = (s_rows0, s_rows1)
        r_rows = (r_rows0, r_rows1)
        c_rows = (c_rows0, c_rows1)
        d_rows = (d_rows0, d_rows1)
        semI = (semI0, semI1)
        semR = (semR0, semR1)

        pltpu.sync_copy(wa_h, wav)
        pltpu.sync_copy(wab_h, wabv)
        wa_regs = [wav[pl.ds(16 * j, 16)] for j in range(4)]
        wab0 = wabv[...][0]
        lane = lax.iota(jnp.int32, 16)

        idx_groups = ((sub_h, subv), (rel_h, relv), (tau_h, tauv),
                      (ridx_h, ridxv))

        def issue_idx(i, p):
            base = (w + i * nw) * B
            for hbm, bufs in idx_groups:
                pltpu.async_copy(hbm.at[pl.ds(base, B)], bufs[p], semI[p])

        def wait_idx(p):
            for hbm, bufs in idx_groups:
                pltpu.make_async_copy(hbm.at[pl.ds(0, B)], bufs[p],
                                      semI[p]).wait()

        def issue_rows(p):
            pltpu.async_copy(as_h.at[subv[p]], s_rows[p], semR[p])
            pltpu.async_copy(br_h.at[relv[p]], r_rows[p], semR[p])
            pltpu.async_copy(dt_h.at[tauv[p]], d_rows[p], semR[p])
            pltpu.async_copy(cq_h.at[ridxv[p]], c_rows[p], semR[p])

        def wait_rows(p):
            pltpu.make_async_copy(as_h.at[subv[p]], s_rows[p], semR[p]).wait()
            pltpu.make_async_copy(br_h.at[relv[p]], r_rows[p], semR[p]).wait()
            pltpu.make_async_copy(dt_h.at[tauv[p]], d_rows[p], semR[p]).wait()
            pltpu.make_async_copy(cq_h.at[ridxv[p]], c_rows[p], semR[p]).wait()

        def compute(i, p):
            base = (w + i * nw) * B
            sr, rr, cr, dr = s_rows[p], r_rows[p], c_rows[p], d_rows[p]

            def group(g, gcarry):
                zv = jnp.zeros((16,), F32)
                for k in range(16):
                    e = g * 16 + k
                    vacc = None
                    for j in range(4):
                        dj = pl.ds(16 * j, 16)
                        y = sr[e, dj] + rr[e, dj] + cr[e, dj] + dr[e, dj]
                        y = jnp.maximum(y, 0.0) * wa_regs[j]
                        vacc = y if vacc is None else vacc + y
                    zv = jnp.where(lane == k, jnp.sum(vacc), zv)
                abuf[pl.ds(g * 16, 16)] = zv + wab0
                return gcarry

            lax.fori_loop(0, B // 16, group, 0)
            pltpu.sync_copy(abuf, alpha_o.at[pl.ds(base, B)])

        def pipe_body(i, p):
            q = 1 - p

            @pl.when(i + 1 < nblk_w)
            def _():
                wait_idx(q)
                issue_rows(q)

            wait_rows(p)
            compute(i, p)

            @pl.when(i + 2 < nblk_w)
            def _():
                issue_idx(i + 2, p)

        issue_idx(0, 0)
        issue_idx(1, 1)
        wait_idx(0)
        issue_rows(0)

        def step(t, carry):
            pipe_body(t * 2, 0)
            pipe_body(t * 2 + 1, 1)
            return carry

        lax.fori_loop(0, nblk_w // 2, step, 0)

    return alpha_kernel(as_t, br_t, cq_t, dt_t, sub, rel, tau, ridx, wa, wab)


def _aggregate_sc(hid4, rela4, ht4, alpha, sub4, rel4, tau4, obj, zrows, n, np_):
    ep = obj.shape[0]
    nblk = ep // B
    nblk_w = nblk // NS
    rows_per_sub = np_ // NS
    mesh = plsc.VectorSubcoreMesh(core_axis_name="c", subcore_axis_name="s",
                                  num_cores=NC, num_subcores=NS)

    @functools.partial(
        pl.kernel,
        out_type=(
            jax.ShapeDtypeStruct((4, np_, 32), F32),
            jax.ShapeDtypeStruct((4, np_, 32), F32),
        ),
        mesh=mesh,
        scratch_types=[
            pltpu.VMEM((B,), jnp.int32),
            pltpu.VMEM((B,), jnp.int32),
            pltpu.VMEM((B,), jnp.int32),
            pltpu.VMEM((B,), jnp.int32),
            pltpu.VMEM((B,), jnp.int32),
            pltpu.VMEM((B,), jnp.int32),
            pltpu.VMEM((B,), jnp.int32),
            pltpu.VMEM((B,), jnp.int32),
            pltpu.VMEM((B,), F32),
            pltpu.VMEM((B,), F32),
            pltpu.VMEM((B, 32), F32),
            pltpu.VMEM((B, 32), F32),
            pltpu.VMEM((B, 32), F32),
            pltpu.VMEM((B, 32), F32),
            pltpu.VMEM((B, 32), F32),
            pltpu.VMEM((B, 32), F32),
            pltpu.VMEM((B, 32), F32),
            pltpu.VMEM((B, 32), F32),
            pltpu.VMEM((B, 32), F32),
            pltpu.VMEM((B, 32), F32),
            pltpu.VMEM((rows_per_sub, 32), F32),
            pltpu.VMEM_SHARED((np_, 32), F32),
            pltpu.VMEM_SHARED((np_, 32), F32),
            pltpu.SemaphoreType.DMA,
            pltpu.SemaphoreType.DMA,
            pltpu.SemaphoreType.DMA,
            pltpu.SemaphoreType.DMA,
            pltpu.SemaphoreType.DMA,
            pltpu.SemaphoreType.DMA,
        ],
        compiler_params=pltpu.CompilerParams(needs_layout_passes=False,
                                             use_tc_tiling_on_sc=False),
    )
    def agg_kernel(hid_h, rela_h, ht_h, alpha_h, sub_h, rel_h, tau_h, obj_h,
                   z_h, accw_o, acct_o,
                   subv0, subv1, relv0, relv1, tauv0, tauv1, objv0, objv1,
                   av0, av1, s_rows0, s_rows1, r_rows0, r_rows1,
                   t_rows0, t_rows1, mw0, mw1, mt0, mt1, zv, accw, acct,
                   semI0, semI1, semR0, semR1, semS0, semS1):
        cid = lax.axis_index("c")
        sid = lax.axis_index("s")
        subv = (subv0, subv1)
        relv = (relv0, relv1)
        tauv = (tauv0, tauv1)
        objv = (objv0, objv1)
        av = (av0, av1)
        s_rows = (s_rows0, s_rows1)
        r_rows = (r_rows0, r_rows1)
        t_rows = (t_rows0, t_rows1)
        mw = (mw0, mw1)
        mt = (mt0, mt1)
        semI = (semI0, semI1)
        semR = (semR0, semR1)
        semS = (semS0, semS1)

        pltpu.sync_copy(z_h, zv)
        sl = pl.ds(sid * rows_per_sub, rows_per_sub)

        for ph in range(2):
            qq = cid * 2 + ph

            pltpu.sync_copy(zv, accw.at[sl])
            pltpu.sync_copy(zv, acct.at[sl])
            plsc.subcore_barrier()

            def issue_idx(i, p):
                ebase = qq * ep + (sid + i * NS) * B
                pltpu.async_copy(sub_h.at[pl.ds(ebase, B)], subv[p], semI[p])
                pltpu.async_copy(rel_h.at[pl.ds(ebase, B)], relv[p], semI[p])
                pltpu.async_copy(tau_h.at[pl.ds(ebase, B)], tauv[p], semI[p])

            def wait_idx(p):
                pltpu.make_async_copy(sub_h.at[pl.ds(0, B)], subv[p], semI[p]).wait()
                pltpu.make_async_copy(rel_h.at[pl.ds(0, B)], relv[p], semI[p]).wait()
                pltpu.make_async_copy(tau_h.at[pl.ds(0, B)], tauv[p], semI[p]).wait()

            def issue_rows(i, p):
                base = (sid + i * NS) * B
                pltpu.async_copy(hid_h.at[subv[p]], s_rows[p], semR[p])
                pltpu.async_copy(rela_h.at[relv[p]], r_rows[p], semR[p])
                pltpu.async_copy(ht_h.at[tauv[p]], t_rows[p], semR[p])
                pltpu.async_copy(obj_h.at[pl.ds(base, B)], objv[p], semR[p])
                pltpu.async_copy(alpha_h.at[pl.ds(base, B)], av[p], semR[p])

            def wait_rows(p):
                pltpu.make_async_copy(hid_h.at[subv[p]], s_rows[p], semR[p]).wait()
                pltpu.make_async_copy(rela_h.at[relv[p]], r_rows[p], semR[p]).wait()
                pltpu.make_async_copy(ht_h.at[tauv[p]], t_rows[p], semR[p]).wait()
                pltpu.make_async_copy(obj_h.at[pl.ds(0, B)], objv[p], semR[p]).wait()
                pltpu.make_async_copy(alpha_h.at[pl.ds(0, B)], av[p], semR[p]).wait()

            def issue_scat(p):
                pltpu.async_copy(mw[p], accw.at[objv[p]], semS[p], add=True)
                pltpu.async_copy(mt[p], acct.at[objv[p]], semS[p], add=True)

            def wait_scat(p):
                pltpu.make_async_copy(mw[p], accw.at[objv[p]], semS[p]).wait()
                pltpu.make_async_copy(mt[p], acct.at[objv[p]], semS[p]).wait()

            def compute(i, p):
                sr, rr, tr = s_rows[p], r_rows[p], t_rows[p]
                avp = av[p]
                mwp, mtp = mw[p], mt[p]

                def group(g, gcarry):
                    a16 = avp[pl.ds(g * 16, 16)]
                    for k in range(16):
                        e = g * 16 + k
                        a = a16[k]
                        for j in range(2):
                            dj = pl.ds(16 * j, 16)
                            m = sr[e, dj] + rr[e, dj] + tr[e, dj]
                            am = a * m
                            mwp[e, dj] = am
                            mtp[e, dj] = m - am
                    return gcarry

                lax.fori_loop(0, B // 16, group, 0)

            def pipe_body(i, p):
                q = 1 - p

                @pl.when(i + 1 < nblk_w)
                def _():
                    wait_idx(q)

                @pl.when((i >= 1) & (i + 1 < nblk_w))
                def _():
                    wait_scat(q)

                @pl.when(i + 1 < nblk_w)
                def _():
                    issue_rows(i + 1, q)

                wait_rows(p)
                compute(i, p)
                issue_scat(p)

                @pl.when(i + 2 < nblk_w)
                def _():
                    issue_idx(i + 2, p)

            issue_idx(0, 0)
            issue_idx(1, 1)
            wait_idx(0)
            issue_rows(0, 0)

            def pstep(t, carry):
                pipe_body(t * 2, 0)
                pipe_body(t * 2 + 1, 1)
                return carry

            lax.fori_loop(0, nblk_w // 2, pstep, 0)
            wait_scat(0)
            wait_scat(1)

            plsc.subcore_barrier()
            pltpu.sync_copy(accw.at[sl], accw_o.at[qq, sl])
            pltpu.sync_copy(acct.at[sl], acct_o.at[qq, sl])

    return agg_kernel(hid4, rela4, ht4, alpha, sub4, rel4, tau4, obj, zrows)


def _final_tc(accw, acct, wh, whs, n, np_):
    def body(aw_r, at_r, wh_r, whs_r, o1, o2):
        o1[...] = jnp.dot(aw_r[...], wh_r[...], preferred_element_type=F32,
                          precision=_HP)[:n]
        o2[...] = jnp.dot(at_r[...], whs_r[...], preferred_element_type=F32,
                          precision=_HP)[:n]

    return pl.pallas_call(
        body,
        out_shape=(
            jax.ShapeDtypeStruct((n, 128), F32),
            jax.ShapeDtypeStruct((n, 128), F32),
        ),
        compiler_params=pltpu.CompilerParams(vmem_limit_bytes=100 * 1024 * 1024),
    )(accw, acct, wh, whs)


def kernel(q_sub, q_rel, q_tau, hidden, edges, n_node, old_nodes_new_idx,
           rela_embed, Ws_attn, Wr_attn, Wqr_attn_w, Wqr_attn_b, Wtau_attn,
           w_alpha_w, w_alpha_b, W_h, W_h_s, weight_t1, bias_t1, weight_t2,
           bias_t2):
    n, d = hidden.shape
    nr = rela_embed.shape[0]
    e = edges.shape[0]
    nwk = NC * NS
    blk_unit = 2 * B * nwk
    ep = ((e + blk_unit - 1) // blk_unit) * blk_unit
    np_ = ((n + NS - 1) // NS) * NS + NS
    qp = ((q_rel.shape[0] + B * NS - 1) // (B * NS)) * (B * NS)

    ed = edges.astype(jnp.int32)
    qt = jnp.asarray(q_tau, jnp.int32)

    def padi(x, value):
        return jnp.concatenate(
            [x, jnp.full((ep - e,), value, jnp.int32)]) if ep > e else x

    sub = padi(ed[:, 5], 0)
    rel = padi(ed[:, 2], 0)
    obj = padi(ed[:, 6], n)
    ridx = padi(ed[:, 0], 0)
    tau_raw = ed[:, 4]
    tau = padi(jnp.where(tau_raw >= 0, tau_raw, qt), 0)
    qrel = jnp.concatenate(
        [q_rel.astype(jnp.int32),
         jnp.zeros((qp - q_rel.shape[0],), jnp.int32)])

    delta = (jnp.arange(n, dtype=jnp.int32) - qt).astype(F32)[:, None]
    as_t, br_t, cf_t, dt_t, ht_t = _tables_tc(
        hidden, rela_embed, delta, Ws_attn, Wr_attn, Wqr_attn_w,
        Wqr_attn_b.reshape(1, 64), Wtau_attn, weight_t1, bias_t1,
        weight_t2, bias_t2)

    cq_t = _cq_sc(cf_t, qrel)
    wa = w_alpha_w.reshape(64)
    wab = jnp.broadcast_to(w_alpha_b.reshape(1), (16,))
    zlogit = _alpha_sc(as_t, br_t, cq_t, dt_t, sub, rel, tau, ridx, wa, wab)
    alpha = _sigmoid_tc(zlogit.reshape(ep // 128, 128)).reshape(ep)

    hid4 = jnp.concatenate([hidden[:, 32 * q:32 * (q + 1)] for q in range(4)],
                           axis=0)
    rela4 = jnp.concatenate(
        [rela_embed[:, 32 * q:32 * (q + 1)] for q in range(4)], axis=0)
    ht4 = jnp.concatenate([ht_t[:, 32 * q:32 * (q + 1)] for q in range(4)],
                          axis=0)
    sub4 = jnp.concatenate([sub + q * n for q in range(4)])
    rel4 = jnp.concatenate([rel + q * nr for q in range(4)])
    tau4 = jnp.concatenate([tau + q * n for q in range(4)])
    zrows = jnp.zeros((np_ // NS, 32), F32)
    accw, acct = _aggregate_sc(hid4, rela4, ht4, alpha, sub4, rel4, tau4,
                               obj, zrows, n, np_)

    accw_f = jnp.moveaxis(accw, 0, 1).reshape(np_, 128)
    acct_f = jnp.moveaxis(acct, 0, 1).reshape(np_, 128)
    hidden_new, hidden_new_s = _final_tc(accw_f, acct_f, W_h, W_h_s, n, np_)
    return (hidden_new, hidden_new_s)
